# baseline (speedup 1.0000x reference)
import functools
import math

import jax
import jax.numpy as jnp
from jax import lax
from jax.experimental import pallas as pl
from jax.experimental.pallas import tpu as pltpu
from jax.experimental.pallas import tpu_sc as plsc

D_MODEL = 64
SCALE = math.sqrt(D_MODEL)

NC = 2
NS = 16
NW = NC * NS
LANES = 16

K = 128
G = 5
C = G * K


def _emb_body(ngather, idx_hbm, table_hbm, out_hbm,
              idx_v, rows0, rows1, g0, g1, s0, s1):
    nstage = ngather // G
    wid = lax.axis_index("s") * NC + lax.axis_index("c")
    base = wid * (nstage * C)
    rows_b = (rows0, rows1)
    gsem = (g0, g1)
    ssem = (s0, s1)

    pltpu.sync_copy(idx_hbm.at[wid], idx_v)

    def issue_gathers(si, b):
        for j in range(G):
            pltpu.async_copy(table_hbm.at[idx_v.at[si * G + j]],
                             rows_b[b].at[pl.ds(j * K, K)], gsem[b])

    def wait_gathers(si, b):
        for j in range(G):
            pltpu.make_async_copy(table_hbm.at[idx_v.at[si * G + j]],
                                  rows_b[b].at[pl.ds(j * K, K)],
                                  gsem[b]).wait()

    def wait_store(b):
        pltpu.make_async_copy(rows_b[b], out_hbm.at[pl.ds(base, C)],
                              ssem[b]).wait()

    issue_gathers(0, 0)

    @pl.loop(0, nstage, step=2)
    def _(i):
        for b in range(2):
            ci = i + b
            nb = 1 - b

            @pl.when(ci + 1 < nstage)
            def _():
                @pl.when(ci >= 1)
                def _():
                    wait_store(nb)
                issue_gathers(ci + 1, nb)

            wait_gathers(ci, b)

            @pl.loop(0, C)
            def _(r):
                for c4 in range(D_MODEL // LANES):
                    sl = pl.ds(c4 * LANES, LANES)
                    rows_b[b][r, sl] = rows_b[b][r, sl] * SCALE

            pltpu.async_copy(rows_b[b], out_hbm.at[pl.ds(base + ci * C, C)],
                             ssem[b])

    wait_store(0)
    wait_store(1)


@jax.jit
def _emb(xf, table):
    n = xf.shape[0] * xf.shape[1] * xf.shape[2]
    ngather = xf.shape[1]
    mesh = plsc.VectorSubcoreMesh(core_axis_name="c", subcore_axis_name="s")
    body = functools.partial(_emb_body, ngather)
    return pl.kernel(
        body,
        out_type=jax.ShapeDtypeStruct((n, D_MODEL), jnp.float32),
        mesh=mesh,
        compiler_params=pltpu.CompilerParams(use_tc_tiling_on_sc=False),
        scratch_types=[
            pltpu.VMEM((ngather, K), jnp.int32),
            pltpu.VMEM((C, D_MODEL), jnp.float32),
            pltpu.VMEM((C, D_MODEL), jnp.float32),
            pltpu.SemaphoreType.DMA,
            pltpu.SemaphoreType.DMA,
            pltpu.SemaphoreType.DMA,
            pltpu.SemaphoreType.DMA,
        ],
    )(xf, table)


def kernel(x, table):
    b, s = x.shape
    n = b * s
    assert n % (NW * C) == 0, (n, NW, C)
    ngather = n // (NW * K)
    xf = x.reshape(NW, ngather, K).astype(jnp.int32)
    out = _emb(xf, table)
    return out.reshape(b, s, D_MODEL)
